# baseline (speedup 1.0000x reference)
import functools

import jax
import jax.numpy as jnp
from jax import lax
from jax.experimental import pallas as pl
from jax.experimental.pallas import tpu as pltpu


def _gemm_kernel(x_ref, wa_ref, wb2_ref, o_ref, wb_ref):
    K = wb_ref.shape[0]
    h = K // 2

    @pl.when(pl.program_id(1) == 0)
    def _cast_weight():
        wb_ref[:h, :] = wa_ref[0].astype(jnp.bfloat16)
        wb_ref[h:, :] = wb2_ref[0].astype(jnp.bfloat16)

    x_bf = x_ref[...].astype(jnp.bfloat16)
    o_ref[:, :h] = lax.dot_general(
        x_bf,
        wb_ref[:h, :],
        dimension_numbers=(((1,), (1,)), ((), ())),
        preferred_element_type=jnp.float32,
    )
    o_ref[:, h:] = lax.dot_general(
        x_bf,
        wb_ref[h:, :],
        dimension_numbers=(((1,), (1,)), ((), ())),
        preferred_element_type=jnp.float32,
    )


@functools.partial(jax.jit, static_argnames=("tm",))
def _solve(xp, m_param, tm=512):
    B, M, N = xp.shape
    K = m_param.shape[0]
    rows = B * M
    x2d = xp.reshape(rows, N)

    tm = min(tm, rows)
    grid_m = pl.cdiv(rows, tm)
    inner = grid_m // 2 if grid_m % 2 == 0 else grid_m
    outer = grid_m // inner

    w3 = m_param.reshape(2, K // 2, N)

    out = pl.pallas_call(
        _gemm_kernel,
        out_shape=jax.ShapeDtypeStruct((rows, K), jnp.float32),
        grid=(outer, inner),
        in_specs=[
            pl.BlockSpec((tm, N), lambda i, j: (i * inner + j, 0)),
            pl.BlockSpec((1, K // 2, N), lambda i, j: (0, 0, 0)),
            pl.BlockSpec((1, K // 2, N), lambda i, j: (1, 0, 0)),
        ],
        out_specs=pl.BlockSpec((tm, K), lambda i, j: (i * inner + j, 0)),
        scratch_shapes=[pltpu.VMEM((K, N), jnp.bfloat16)],
        compiler_params=pltpu.CompilerParams(
            dimension_semantics=("parallel", "arbitrary"),
            vmem_limit_bytes=56 << 20,
        ),
    )(x2d, w3, w3)
    return out.reshape(B, M, K)


def kernel(xp, m_param):
    return _solve(xp, m_param)


# speedup vs baseline: 1.0024x; 1.0024x over previous
import functools

import jax
import jax.numpy as jnp
from jax import lax
from jax.experimental import pallas as pl
from jax.experimental.pallas import tpu as pltpu


def _gemm_kernel(x_ref, wa_ref, wb2_ref, o_ref, wb_ref, xb_ref):
    K = wb_ref.shape[0]
    h = K // 2

    @pl.when(pl.program_id(1) == 0)
    def _cast_weight():
        wb_ref[:h, :] = wa_ref[0].astype(jnp.bfloat16)
        wb_ref[h:, :] = wb2_ref[0].astype(jnp.bfloat16)

    xb_ref[...] = x_ref[...].astype(jnp.bfloat16)
    o_ref[:, :h] = lax.dot_general(
        xb_ref[...],
        wb_ref[:h, :],
        dimension_numbers=(((1,), (1,)), ((), ())),
        preferred_element_type=jnp.float32,
    )
    o_ref[:, h:] = lax.dot_general(
        xb_ref[...],
        wb_ref[h:, :],
        dimension_numbers=(((1,), (1,)), ((), ())),
        preferred_element_type=jnp.float32,
    )


@functools.partial(jax.jit, static_argnames=("tm",))
def _solve(xp, m_param, tm=512):
    B, M, N = xp.shape
    K = m_param.shape[0]
    rows = B * M
    x2d = xp.reshape(rows, N)

    tm = min(tm, rows)
    grid_m = pl.cdiv(rows, tm)
    inner = grid_m // 2 if grid_m % 2 == 0 else grid_m
    outer = grid_m // inner

    w3 = m_param.reshape(2, K // 2, N)

    out = pl.pallas_call(
        _gemm_kernel,
        out_shape=jax.ShapeDtypeStruct((rows, K), jnp.float32),
        grid=(outer, inner),
        in_specs=[
            pl.BlockSpec((tm, N), lambda i, j: (i * inner + j, 0)),
            pl.BlockSpec((1, K // 2, N), lambda i, j: (0, 0, 0)),
            pl.BlockSpec((1, K // 2, N), lambda i, j: (1, 0, 0)),
        ],
        out_specs=pl.BlockSpec((tm, K), lambda i, j: (i * inner + j, 0)),
        scratch_shapes=[
            pltpu.VMEM((K, N), jnp.bfloat16),
            pltpu.VMEM((tm, N), jnp.bfloat16),
        ],
        compiler_params=pltpu.CompilerParams(
            dimension_semantics=("parallel", "arbitrary"),
            vmem_limit_bytes=56 << 20,
        ),
    )(x2d, w3, w3)
    return out.reshape(B, M, K)


def kernel(xp, m_param):
    return _solve(xp, m_param)
